# baseline (speedup 1.0000x reference)
import functools

import jax
import jax.numpy as jnp
from jax import lax
from jax.experimental import pallas as pl
from jax.experimental.pallas import tpu as pltpu
from jax.experimental.pallas import tpu_sc as plsc

_N = 262144
_F = 256
_SEL = tuple(range(0, _F, 8))

_NC = 2
_NS = 16
_NW = _NC * _NS

_N_SC = 0
_N_TC = _N - _N_SC

_RPW = _N_SC // _NW
_CHUNK = 64
_NCHUNK = _RPW // _CHUNK
_NBUF = 4
_L = 16

_BN = 1024


def _sc_body(x_hbm, thr_hbm, out_hbm, *refs):
    bufs = refs[:_NBUF]
    out_v, thr_v = refs[_NBUF], refs[_NBUF + 1]
    sems = refs[_NBUF + 2:]
    wid = lax.axis_index("s") * _NC + lax.axis_index("c")
    row0 = wid * _RPW
    pltpu.sync_copy(thr_hbm, thr_v)
    lanes = lax.iota(jnp.int32, _L)

    tvecs = []
    for c in _SEL:
        grp = thr_v[pl.ds((c // _L) * _L, _L)]
        tvecs.append(jnp.full((_L,), grp[c % _L], jnp.float32))

    def start_fetch(i, b):
        base = row0 + i * _CHUNK
        pltpu.async_copy(x_hbm.at[pl.ds(base, _CHUNK), :], bufs[b], sems[b])

    def compute_chunk(i, b):
        buf = bufs[b]

        def group_body(g, carry2):
            rows = g * _L + lanes
            acc = jnp.full((_L,), jnp.inf, jnp.float32)
            for j, c in enumerate(_SEL):
                cols = jnp.full((_L,), c, jnp.int32)
                v = plsc.load_gather(buf, [rows, cols])
                acc = jnp.minimum(acc, v - tvecs[j])
            res = jnp.where(acc > 0.0, jnp.int32(1), jnp.int32(0))
            out_v[pl.ds(i * _CHUNK + g * _L, _L)] = res
            return carry2

        lax.fori_loop(0, _CHUNK // _L, group_body, 0, unroll=False)

    for b in range(_NBUF - 1):
        start_fetch(b, b)

    def ring_body(p, carry):
        for b in range(_NBUF):
            i = _NBUF * p + b
            pltpu.make_async_copy(
                x_hbm.at[pl.ds(0, _CHUNK), :], bufs[b], sems[b]
            ).wait()
            nxt = i + _NBUF - 1

            @pl.when(nxt < _NCHUNK)
            def _():
                start_fetch(nxt, (b + _NBUF - 1) % _NBUF)

            compute_chunk(i, b)

        return carry

    lax.fori_loop(0, _NCHUNK // _NBUF, ring_body, 0, unroll=False)
    pltpu.sync_copy(out_v, out_hbm.at[pl.ds(row0, _RPW)])


def _match_sc(x, thresholds):
    mesh = plsc.VectorSubcoreMesh(core_axis_name="c", subcore_axis_name="s")
    run = pl.kernel(
        _sc_body,
        out_type=jax.ShapeDtypeStruct((_N_SC,), jnp.int32),
        mesh=mesh,
        compiler_params=pltpu.CompilerParams(needs_layout_passes=False),
        scratch_types=(
            [pltpu.VMEM((_CHUNK, _F), jnp.float32)] * _NBUF
            + [
                pltpu.VMEM((_RPW,), jnp.int32),
                pltpu.VMEM((_F,), jnp.float32),
            ]
            + [pltpu.SemaphoreType.DMA] * _NBUF
        ),
    )
    return run(x, thresholds)


def _tc_body(x_ref, thr_ref, out_ref):
    xb = x_ref[...]
    d = xb - thr_ref[0:1, :]
    col = lax.broadcasted_iota(jnp.int32, (_BN, _F), 1)
    vals = jnp.where(col % 8 == 0, d, jnp.inf)
    rowmin = jnp.min(vals, axis=1, keepdims=True)
    out_ref[...] = (rowmin > 0.0).astype(jnp.int32)


def _match_tc(x, thresholds):
    thr2d = jnp.broadcast_to(thresholds[None, :], (8, _F))
    off_b = _N_SC // _BN
    out = pl.pallas_call(
        _tc_body,
        grid=(_N_TC // _BN,),
        in_specs=[
            pl.BlockSpec((_BN, _F), lambda i: (off_b + i, 0)),
            pl.BlockSpec((8, _F), lambda i: (0, 0)),
        ],
        out_specs=pl.BlockSpec((_BN, 1), lambda i: (i, 0)),
        out_shape=jax.ShapeDtypeStruct((_N_TC, 1), jnp.int32),
        compiler_params=pltpu.CompilerParams(
            dimension_semantics=("arbitrary",)
        ),
    )(x, thr2d)
    return out.reshape(_N_TC)


@jax.jit
def _match(x, thresholds):
    parts = []
    if _N_SC > 0:
        parts.append(_match_sc(x, thresholds))
    if _N_TC > 0:
        parts.append(_match_tc(x, thresholds))
    out = parts[0] if len(parts) == 1 else jnp.concatenate(parts)
    return out.astype(jnp.bool_)


def kernel(x, thresholds):
    return _match(x, thresholds)


# speedup vs baseline: 1.9591x; 1.9591x over previous
import functools

import jax
import jax.numpy as jnp
from jax import lax
from jax.experimental import pallas as pl
from jax.experimental.pallas import tpu as pltpu
from jax.experimental.pallas import tpu_sc as plsc

_N = 262144
_F = 256
_SEL = tuple(range(0, _F, 8))

_NC = 2
_NS = 16
_NW = _NC * _NS

_N_SC = 172032
_N_TC = _N - _N_SC

_RPW = _N_SC // _NW
_CHUNK = 64
_NCHUNK = _RPW // _CHUNK
_NBUF = 4
_L = 16

_BN = 1024


def _sc_body(x_hbm, thr_hbm, out_hbm, *refs):
    bufs = refs[:_NBUF]
    out_v, thr_v = refs[_NBUF], refs[_NBUF + 1]
    sems = refs[_NBUF + 2:]
    wid = lax.axis_index("s") * _NC + lax.axis_index("c")
    row0 = wid * _RPW
    pltpu.sync_copy(thr_hbm, thr_v)
    lanes = lax.iota(jnp.int32, _L)

    tvecs = []
    for c in _SEL:
        grp = thr_v[pl.ds((c // _L) * _L, _L)]
        tvecs.append(jnp.full((_L,), grp[c % _L], jnp.float32))

    def start_fetch(i, b):
        base = row0 + i * _CHUNK
        pltpu.async_copy(x_hbm.at[pl.ds(base, _CHUNK), :], bufs[b], sems[b])

    def compute_chunk(i, b):
        buf = bufs[b]

        def group_body(g, carry2):
            rows = g * _L + lanes
            acc = jnp.full((_L,), jnp.inf, jnp.float32)
            for j, c in enumerate(_SEL):
                cols = jnp.full((_L,), c, jnp.int32)
                v = plsc.load_gather(buf, [rows, cols])
                acc = jnp.minimum(acc, v - tvecs[j])
            res = jnp.where(acc > 0.0, jnp.int32(1), jnp.int32(0))
            out_v[pl.ds(i * _CHUNK + g * _L, _L)] = res
            return carry2

        lax.fori_loop(0, _CHUNK // _L, group_body, 0, unroll=False)

    for b in range(_NBUF - 1):
        start_fetch(b, b)

    def ring_body(p, carry):
        for b in range(_NBUF):
            i = _NBUF * p + b
            pltpu.make_async_copy(
                x_hbm.at[pl.ds(0, _CHUNK), :], bufs[b], sems[b]
            ).wait()
            nxt = i + _NBUF - 1

            @pl.when(nxt < _NCHUNK)
            def _():
                start_fetch(nxt, (b + _NBUF - 1) % _NBUF)

            compute_chunk(i, b)

        return carry

    lax.fori_loop(0, _NCHUNK // _NBUF, ring_body, 0, unroll=False)
    pltpu.sync_copy(out_v, out_hbm.at[pl.ds(row0, _RPW)])


def _match_sc(x, thresholds):
    mesh = plsc.VectorSubcoreMesh(core_axis_name="c", subcore_axis_name="s")
    run = pl.kernel(
        _sc_body,
        out_type=jax.ShapeDtypeStruct((_N_SC,), jnp.int32),
        mesh=mesh,
        compiler_params=pltpu.CompilerParams(needs_layout_passes=False),
        scratch_types=(
            [pltpu.VMEM((_CHUNK, _F), jnp.float32)] * _NBUF
            + [
                pltpu.VMEM((_RPW,), jnp.int32),
                pltpu.VMEM((_F,), jnp.float32),
            ]
            + [pltpu.SemaphoreType.DMA] * _NBUF
        ),
    )
    return run(x, thresholds)


def _tc_body(x_ref, thr_ref, out_ref):
    xb = x_ref[...]
    d = xb - thr_ref[0:1, :]
    col = lax.broadcasted_iota(jnp.int32, (_BN, _F), 1)
    vals = jnp.where(col % 8 == 0, d, jnp.inf)
    rowmin = jnp.min(vals, axis=1, keepdims=True)
    out_ref[...] = (rowmin > 0.0).astype(jnp.int32)


def _match_tc(x, thresholds):
    thr2d = jnp.broadcast_to(thresholds[None, :], (8, _F))
    off_b = _N_SC // _BN
    out = pl.pallas_call(
        _tc_body,
        grid=(_N_TC // _BN,),
        in_specs=[
            pl.BlockSpec((_BN, _F), lambda i: (off_b + i, 0)),
            pl.BlockSpec((8, _F), lambda i: (0, 0)),
        ],
        out_specs=pl.BlockSpec((_BN, 1), lambda i: (i, 0)),
        out_shape=jax.ShapeDtypeStruct((_N_TC, 1), jnp.int32),
        compiler_params=pltpu.CompilerParams(
            dimension_semantics=("arbitrary",)
        ),
    )(x, thr2d)
    return out.reshape(_N_TC)


@jax.jit
def _match(x, thresholds):
    parts = []
    if _N_SC > 0:
        parts.append(_match_sc(x, thresholds))
    if _N_TC > 0:
        parts.append(_match_tc(x, thresholds))
    out = parts[0] if len(parts) == 1 else jnp.concatenate(parts)
    return out.astype(jnp.bool_)


def kernel(x, thresholds):
    return _match(x, thresholds)
